# baseline (speedup 1.0000x reference)
import jax
import jax.numpy as jnp
from jax import lax
from jax.experimental import pallas as pl
from jax.experimental.pallas import tpu as pltpu
from jax.experimental.pallas import tpu_sc as plsc

VOCAB = 100000
NCOMP = 2
EMBED = 64
B = 16384
NC = 2
NS = 16
NW = NC * NS
BPW = B // NW
CHUNK = 128
NCHUNK = BPW // CHUNK

VCH = 12288
NBLK = (VOCAB + VCH - 1) // VCH


def _plane_sum_body(t0, t1, t2, t3, mix, out):
    i = pl.program_id(0)
    s = (t0[...].reshape(NCOMP * EMBED, VCH).sum(axis=0)
         + t1[...].reshape(NCOMP * EMBED, VCH).sum(axis=0)
         + t2[...].reshape(NCOMP * EMBED, VCH).sum(axis=0)
         + t3[...].reshape(NCOMP * EMBED, VCH).sum(axis=0)
         + mix[...].sum(axis=0))
    pos = i * VCH + lax.broadcasted_iota(jnp.int32, (VCH,), 0)
    out[...] = jnp.where(pos < VOCAB, s, 0.0)


def _gather_sum_body(widx_hbm, p_hbm, out_hbm, idx_v, g_v, outbuf, sem):
    wid = lax.axis_index("s") * NC + lax.axis_index("c")
    base = wid * BPW
    pltpu.sync_copy(widx_hbm.at[pl.ds(base, BPW)], idx_v)

    handles = [
        pltpu.async_copy(
            p_hbm.at[idx_v.at[pl.ds(ck * CHUNK, CHUNK)]],
            g_v.at[pl.ds(ck * CHUNK, CHUNK)],
            sem,
        )
        for ck in range(NCHUNK)
    ]
    for h in handles:
        h.wait()

    def body(i, m):
        return m + g_v[pl.ds(i * 16, 16)]

    outbuf[...] = lax.fori_loop(0, BPW // 16, body,
                                jnp.zeros((16,), jnp.float32))
    pltpu.sync_copy(outbuf, out_hbm.at[wid])


@jax.jit
def _run(word_idxs, tv0, tv1, tv2, tv3, mixv):
    p = pl.pallas_call(
        _plane_sum_body,
        grid=(NBLK,),
        in_specs=[
            pl.BlockSpec((NCOMP, EMBED, VCH), lambda i: (0, 0, i)),
            pl.BlockSpec((NCOMP, EMBED, VCH), lambda i: (0, 0, i)),
            pl.BlockSpec((NCOMP, EMBED, VCH), lambda i: (0, 0, i)),
            pl.BlockSpec((NCOMP, EMBED, VCH), lambda i: (0, 0, i)),
            pl.BlockSpec((NCOMP, VCH), lambda i: (0, i)),
        ],
        out_specs=pl.BlockSpec((VCH,), lambda i: (i,)),
        out_shape=jax.ShapeDtypeStruct((NBLK * VCH,), jnp.float32),
    )(tv0, tv1, tv2, tv3, mixv)

    mesh = plsc.VectorSubcoreMesh(core_axis_name="c", subcore_axis_name="s")
    f = pl.kernel(
        _gather_sum_body,
        out_type=jax.ShapeDtypeStruct((NW, 16), jnp.float32),
        mesh=mesh,
        scratch_types=[
            pltpu.VMEM((BPW,), jnp.int32),
            pltpu.VMEM((BPW,), jnp.float32),
            pltpu.VMEM((16,), jnp.float32),
            pltpu.SemaphoreType.DMA,
        ],
    )
    partials = f(word_idxs, p)
    return jnp.sum(partials)


def kernel(word_idxs, pos_idxs, neg_idxs, mus, logsigmas, mixture,
           mus_out, logsigmas_out):
    del pos_idxs, neg_idxs
    idx = word_idxs.astype(jnp.int32)
    tv0 = jnp.transpose(mus, (1, 2, 0))
    tv1 = jnp.transpose(logsigmas, (1, 2, 0))
    tv2 = jnp.transpose(mus_out, (1, 2, 0))
    tv3 = jnp.transpose(logsigmas_out, (1, 2, 0))
    mixv = jnp.transpose(mixture, (1, 0))
    return _run(idx, tv0, tv1, tv2, tv3, mixv)


# speedup vs baseline: 1.0074x; 1.0074x over previous
import jax
import jax.numpy as jnp
from jax import lax
from jax.experimental import pallas as pl
from jax.experimental.pallas import tpu as pltpu
from jax.experimental.pallas import tpu_sc as plsc

VOCAB = 100000
NCOMP = 2
EMBED = 64
B = 16384
NC = 2
NS = 16
NW = NC * NS
BPW = B // NW
CHUNK = 128
NCHUNK = BPW // CHUNK

VCH = 10240
NBLK = (VOCAB + VCH - 1) // VCH


def _plane_sum_body(t0, t1, t2, t3, mix, out):
    i = pl.program_id(0)
    s = (t0[...].reshape(NCOMP * EMBED, VCH).sum(axis=0)
         + t1[...].reshape(NCOMP * EMBED, VCH).sum(axis=0)
         + t2[...].reshape(NCOMP * EMBED, VCH).sum(axis=0)
         + t3[...].reshape(NCOMP * EMBED, VCH).sum(axis=0)
         + mix[...].sum(axis=0))
    pos = i * VCH + lax.broadcasted_iota(jnp.int32, (VCH,), 0)
    out[...] = jnp.where(pos < VOCAB, s, 0.0)


def _gather_sum_body(widx_hbm, p_hbm, out_hbm, idx_v, g_v, outbuf, sem):
    wid = lax.axis_index("s") * NC + lax.axis_index("c")
    base = wid * BPW
    pltpu.sync_copy(widx_hbm.at[pl.ds(base, BPW)], idx_v)

    handles = [
        pltpu.async_copy(
            p_hbm.at[idx_v.at[pl.ds(ck * CHUNK, CHUNK)]],
            g_v.at[pl.ds(ck * CHUNK, CHUNK)],
            sem,
        )
        for ck in range(NCHUNK)
    ]
    for h in handles:
        h.wait()

    def body(i, m):
        return m + g_v[pl.ds(i * 16, 16)]

    outbuf[...] = lax.fori_loop(0, BPW // 16, body,
                                jnp.zeros((16,), jnp.float32))
    pltpu.sync_copy(outbuf, out_hbm.at[wid])


@jax.jit
def _run(word_idxs, tv0, tv1, tv2, tv3, mixv):
    p = pl.pallas_call(
        _plane_sum_body,
        grid=(NBLK,),
        in_specs=[
            pl.BlockSpec((NCOMP, EMBED, VCH), lambda i: (0, 0, i)),
            pl.BlockSpec((NCOMP, EMBED, VCH), lambda i: (0, 0, i)),
            pl.BlockSpec((NCOMP, EMBED, VCH), lambda i: (0, 0, i)),
            pl.BlockSpec((NCOMP, EMBED, VCH), lambda i: (0, 0, i)),
            pl.BlockSpec((NCOMP, VCH), lambda i: (0, i)),
        ],
        out_specs=pl.BlockSpec((VCH,), lambda i: (i,)),
        out_shape=jax.ShapeDtypeStruct((NBLK * VCH,), jnp.float32),
    )(tv0, tv1, tv2, tv3, mixv)

    mesh = plsc.VectorSubcoreMesh(core_axis_name="c", subcore_axis_name="s")
    f = pl.kernel(
        _gather_sum_body,
        out_type=jax.ShapeDtypeStruct((NW, 16), jnp.float32),
        mesh=mesh,
        scratch_types=[
            pltpu.VMEM((BPW,), jnp.int32),
            pltpu.VMEM((BPW,), jnp.float32),
            pltpu.VMEM((16,), jnp.float32),
            pltpu.SemaphoreType.DMA,
        ],
    )
    partials = f(word_idxs, p)
    return jnp.sum(partials)


def kernel(word_idxs, pos_idxs, neg_idxs, mus, logsigmas, mixture,
           mus_out, logsigmas_out):
    del pos_idxs, neg_idxs
    idx = word_idxs.astype(jnp.int32)
    tv0 = jnp.transpose(mus, (1, 2, 0))
    tv1 = jnp.transpose(logsigmas, (1, 2, 0))
    tv2 = jnp.transpose(mus_out, (1, 2, 0))
    tv3 = jnp.transpose(logsigmas_out, (1, 2, 0))
    mixv = jnp.transpose(mixture, (1, 0))
    return _run(idx, tv0, tv1, tv2, tv3, mixv)


# speedup vs baseline: 1.0111x; 1.0036x over previous
import jax
import jax.numpy as jnp
from jax import lax
from jax.experimental import pallas as pl
from jax.experimental.pallas import tpu as pltpu
from jax.experimental.pallas import tpu_sc as plsc

VOCAB = 100000
NCOMP = 2
EMBED = 64
B = 16384
NC = 2
NS = 16
NW = NC * NS
BPW = B // NW
CHUNK = 128
NCHUNK = BPW // CHUNK

VCH = 8192
NBLK = (VOCAB + VCH - 1) // VCH


def _plane_sum_body(t0, t1, t2, t3, mix, out):
    i = pl.program_id(0)
    s = (t0[...].reshape(NCOMP * EMBED, VCH).sum(axis=0)
         + t1[...].reshape(NCOMP * EMBED, VCH).sum(axis=0)
         + t2[...].reshape(NCOMP * EMBED, VCH).sum(axis=0)
         + t3[...].reshape(NCOMP * EMBED, VCH).sum(axis=0)
         + mix[...].sum(axis=0))
    pos = i * VCH + lax.broadcasted_iota(jnp.int32, (VCH,), 0)
    out[...] = jnp.where(pos < VOCAB, s, 0.0)


def _gather_sum_body(widx_hbm, p_hbm, out_hbm, idx_v, g_v, outbuf, sem):
    wid = lax.axis_index("s") * NC + lax.axis_index("c")
    base = wid * BPW
    pltpu.sync_copy(widx_hbm.at[pl.ds(base, BPW)], idx_v)

    handles = [
        pltpu.async_copy(
            p_hbm.at[idx_v.at[pl.ds(ck * CHUNK, CHUNK)]],
            g_v.at[pl.ds(ck * CHUNK, CHUNK)],
            sem,
        )
        for ck in range(NCHUNK)
    ]
    for h in handles:
        h.wait()

    def body(i, m):
        return m + g_v[pl.ds(i * 16, 16)]

    outbuf[...] = lax.fori_loop(0, BPW // 16, body,
                                jnp.zeros((16,), jnp.float32))
    pltpu.sync_copy(outbuf, out_hbm.at[wid])


@jax.jit
def _run(word_idxs, tv0, tv1, tv2, tv3, mixv):
    p = pl.pallas_call(
        _plane_sum_body,
        grid=(NBLK,),
        in_specs=[
            pl.BlockSpec((NCOMP, EMBED, VCH), lambda i: (0, 0, i)),
            pl.BlockSpec((NCOMP, EMBED, VCH), lambda i: (0, 0, i)),
            pl.BlockSpec((NCOMP, EMBED, VCH), lambda i: (0, 0, i)),
            pl.BlockSpec((NCOMP, EMBED, VCH), lambda i: (0, 0, i)),
            pl.BlockSpec((NCOMP, VCH), lambda i: (0, i)),
        ],
        out_specs=pl.BlockSpec((VCH,), lambda i: (i,)),
        out_shape=jax.ShapeDtypeStruct((NBLK * VCH,), jnp.float32),
    )(tv0, tv1, tv2, tv3, mixv)

    mesh = plsc.VectorSubcoreMesh(core_axis_name="c", subcore_axis_name="s")
    f = pl.kernel(
        _gather_sum_body,
        out_type=jax.ShapeDtypeStruct((NW, 16), jnp.float32),
        mesh=mesh,
        scratch_types=[
            pltpu.VMEM((BPW,), jnp.int32),
            pltpu.VMEM((BPW,), jnp.float32),
            pltpu.VMEM((16,), jnp.float32),
            pltpu.SemaphoreType.DMA,
        ],
    )
    partials = f(word_idxs, p)
    return jnp.sum(partials)


def kernel(word_idxs, pos_idxs, neg_idxs, mus, logsigmas, mixture,
           mus_out, logsigmas_out):
    del pos_idxs, neg_idxs
    idx = word_idxs.astype(jnp.int32)
    tv0 = jnp.transpose(mus, (1, 2, 0))
    tv1 = jnp.transpose(logsigmas, (1, 2, 0))
    tv2 = jnp.transpose(mus_out, (1, 2, 0))
    tv3 = jnp.transpose(logsigmas_out, (1, 2, 0))
    mixv = jnp.transpose(mixture, (1, 0))
    return _run(idx, tv0, tv1, tv2, tv3, mixv)
